# baseline (speedup 1.0000x reference)
import jax
import jax.numpy as jnp
from jax.experimental import pallas as pl

_BLOCK = 4096


def _router_kernel(x_ref, w1_ref, b1_ref, w2_ref, b2_ref, out_ref):
    x = x_ref[...]
    h = jnp.maximum(
        jnp.dot(x, w1_ref[...], preferred_element_type=jnp.float32) + b1_ref[...],
        0.0,
    )
    s = jnp.dot(h, w2_ref[...], preferred_element_type=jnp.float32) + b2_ref[...]

    e = float(s.shape[1])
    lane = jax.lax.broadcasted_iota(jnp.int32, s.shape, 1).astype(jnp.float32)

    m0 = jnp.max(s, axis=1, keepdims=True)
    i0 = jnp.min(jnp.where(s == m0, lane, e), axis=1, keepdims=True)
    s_masked = jnp.where(lane == i0, -jnp.inf, s)
    m1 = jnp.max(s_masked, axis=1, keepdims=True)
    i1 = jnp.min(jnp.where(s_masked == m1, lane, e), axis=1, keepdims=True)

    denom = jnp.sum(jnp.exp(s - m0), axis=1, keepdims=True)
    p0 = 1.0 / denom
    p1 = jnp.exp(m1 - m0) / denom

    out_ref[...] = jnp.concatenate([i0, i1, p0, p1], axis=1)


def kernel(x, W1, b1, W2, b2):
    n = x.shape[0]
    grid = n // _BLOCK
    packed = pl.pallas_call(
        _router_kernel,
        grid=(grid,),
        in_specs=[
            pl.BlockSpec((_BLOCK, x.shape[1]), lambda i: (i, 0)),
            pl.BlockSpec(W1.shape, lambda i: (0, 0)),
            pl.BlockSpec(b1.shape, lambda i: (0,)),
            pl.BlockSpec(W2.shape, lambda i: (0, 0)),
            pl.BlockSpec(b2.shape, lambda i: (0,)),
        ],
        out_specs=pl.BlockSpec((_BLOCK, 4), lambda i: (i, 0)),
        out_shape=jax.ShapeDtypeStruct((n, 4), jnp.float32),
    )(x, W1, b1, W2, b2)
    idx = packed[:, 0:2].astype(jnp.int32)
    prob = packed[:, 2:4]
    return idx, prob


# speedup vs baseline: 1.2795x; 1.2795x over previous
import jax
import jax.numpy as jnp
from jax.experimental import pallas as pl

_BLOCK = 4096


def _router_kernel(x_ref, w1_ref, b1_ref, w2_ref, b2_ref, idx_ref, prob_ref):
    x = x_ref[...]
    h = jnp.maximum(
        jnp.dot(x, w1_ref[...], preferred_element_type=jnp.float32) + b1_ref[...],
        0.0,
    )
    s = jnp.dot(h, w2_ref[...], preferred_element_type=jnp.float32) + b2_ref[...]

    e = float(s.shape[1])
    lane = jax.lax.broadcasted_iota(jnp.int32, s.shape, 1).astype(jnp.float32)

    m0 = jnp.max(s, axis=1, keepdims=True)
    i0 = jnp.min(jnp.where(s == m0, lane, e), axis=1, keepdims=True)
    s_masked = jnp.where(lane == i0, -jnp.inf, s)
    m1 = jnp.max(s_masked, axis=1, keepdims=True)
    i1 = jnp.min(jnp.where(s_masked == m1, lane, e), axis=1, keepdims=True)

    denom = jnp.sum(jnp.exp(s - m0), axis=1, keepdims=True)
    p0 = 1.0 / denom
    p1 = jnp.exp(m1 - m0) / denom

    idx_ref[...] = jnp.concatenate([i0, i1], axis=1).astype(jnp.int32)
    prob_ref[...] = jnp.concatenate([p0, p1], axis=1)


def kernel(x, W1, b1, W2, b2):
    n = x.shape[0]
    grid = n // _BLOCK
    packed = pl.pallas_call(
        _router_kernel,
        grid=(grid,),
        in_specs=[
            pl.BlockSpec((_BLOCK, x.shape[1]), lambda i: (i, 0)),
            pl.BlockSpec(W1.shape, lambda i: (0, 0)),
            pl.BlockSpec(b1.shape, lambda i: (0,)),
            pl.BlockSpec(W2.shape, lambda i: (0, 0)),
            pl.BlockSpec(b2.shape, lambda i: (0,)),
        ],
        out_specs=[
            pl.BlockSpec((_BLOCK, 2), lambda i: (i, 0)),
            pl.BlockSpec((_BLOCK, 2), lambda i: (i, 0)),
        ],
        out_shape=[
            jax.ShapeDtypeStruct((n, 2), jnp.int32),
            jax.ShapeDtypeStruct((n, 2), jnp.float32),
        ],
    )(x, W1, b1, W2, b2)
    return packed
